# baseline (speedup 1.0000x reference)
import functools

import jax
import jax.numpy as jnp
from jax import lax
from jax.experimental import pallas as pl
from jax.experimental.pallas import tpu as pltpu
from jax.experimental.pallas import tpu_sc as plsc

V = 1000
VP = 1024
NC, NS = 2, 16
NW = NC * NS
B = 1024
L = 50
LP = 64
BPW = B // NW
N = B * L
NVT = V // 8


def _lse_body(t_ref, o_ref):
    x = t_ref[...]
    m = jnp.max(x, axis=1, keepdims=True)
    s = jnp.sum(jnp.exp(x - m), axis=1, keepdims=True)
    o_ref[...] = m + jnp.log(s)


def _loss_body(p_ref, o_ref):
    s = jnp.sum(p_ref[...], axis=1, keepdims=True)
    o_ref[...] = jnp.sum(s, axis=0, keepdims=True) * (1.0 / N)


_mesh = plsc.VectorSubcoreMesh(core_axis_name="c", subcore_axis_name="s")


@functools.partial(
    pl.kernel,
    mesh=_mesh,
    compiler_params=pltpu.CompilerParams(
        use_tc_tiling_on_sc=False, needs_layout_passes=False
    ),
    out_type=[
        jax.ShapeDtypeStruct((L, V, B), jnp.float32),
        jax.ShapeDtypeStruct((NW * 32,), jnp.float32),
    ],
    scratch_types=[
        pltpu.VMEM((N,), jnp.int32),
        pltpu.VMEM((BPW * LP,), jnp.int32),
        pltpu.VMEM((VP,), jnp.float32),
        pltpu.VMEM((8, V), jnp.float32),
        pltpu.VMEM((1, 8, B), jnp.float32),
        pltpu.VMEM((1, 8, B), jnp.float32),
        pltpu.VMEM((BPW * LP,), jnp.int32),
        pltpu.VMEM((BPW * LP,), jnp.float32),
        pltpu.VMEM((16,), jnp.float32),
        pltpu.SemaphoreType.DMA,
        pltpu.SemaphoreType.DMA,
        pltpu.SemaphoreType.DMA,
    ],
)
def _sc_gather(tT_hbm, tflat_hbm, inT_hbm, tgt_hbm, lse_hbm, out_hbm, part_hbm,
               inT_v, tgt_v, lse_v, rowb_v, slab_a, slab_b, fidx_v, pick_v,
               tmp_v, sem_wa, sem_wb, sem_p):
    wid = lax.axis_index("s") * NC + lax.axis_index("c")
    pltpu.sync_copy(inT_hbm, inT_v)
    pltpu.sync_copy(tgt_hbm.at[pl.ds(wid * (BPW * LP), BPW * LP)], tgt_v)
    pltpu.sync_copy(lse_hbm, lse_v)
    lane = lax.iota(jnp.int32, 16)

    def fill_and_write(vt, l, slab, sem_w):
        def fill_j(j, _):
            iv = inT_v[pl.ds(l * B + j * 16, 16)]
            for v_loc in range(8):
                val = plsc.load_gather(
                    rowb_v, [jnp.full((16,), v_loc, jnp.int32), iv])
                slab[0, v_loc, pl.ds(j * 16, 16)] = val
            return 0

        lax.fori_loop(0, B // 16, fill_j, 0)
        pltpu.async_copy(
            slab, out_hbm.at[pl.ds(l, 1), pl.ds(vt * 8, 8)], sem_w)

    def wait_write(sem_w):
        pltpu.make_async_copy(
            slab_a, out_hbm.at[pl.ds(0, 1), pl.ds(0, 8)], sem_w).wait()

    for t in range(4):
        vt = wid + 32 * t

        def vt_block():
            pltpu.sync_copy(tT_hbm.at[pl.ds(vt * 8, 8)], rowb_v)

            def pair(u, _):
                @pl.when(jnp.logical_or(jnp.int32(t) > 0, u > 0))
                def _():
                    wait_write(sem_wa)
                fill_and_write(vt, 2 * u, slab_a, sem_wa)

                @pl.when(jnp.logical_or(jnp.int32(t) > 0, u > 0))
                def _():
                    wait_write(sem_wb)
                fill_and_write(vt, 2 * u + 1, slab_b, sem_wb)
                return 0

            lax.fori_loop(0, L // 2, pair, 0)

        if t < 3:
            vt_block()
        else:
            pl.when(wid < NVT - 96)(vt_block)
    wait_write(sem_wa)
    wait_write(sem_wb)

    b0 = wid * BPW

    def build_fidx(b_loc, _):
        for k in range(4):
            pos = lane + (k * 16)
            cpos = jnp.minimum(pos, L - 1)
            iv = plsc.load_gather(inT_v, [cpos * B + (b0 + b_loc)])
            tv = tgt_v[pl.ds(b_loc * LP + k * 16, 16)]
            fidx_v[pl.ds(b_loc * LP + k * 16, 16)] = iv * V + tv
        return 0

    lax.fori_loop(0, BPW, build_fidx, 0)

    def pick_gather(b_loc, _):
        pltpu.async_copy(
            tflat_hbm.at[fidx_v.at[pl.ds(b_loc * LP, LP)]],
            pick_v.at[pl.ds(b_loc * LP, LP)], sem_p).wait()
        return 0

    lax.fori_loop(0, BPW, pick_gather, 0)

    def accum(b_loc, acc):
        for k in range(4):
            pos = lane + (k * 16)
            cpos = jnp.minimum(pos, L - 1)
            iv = plsc.load_gather(inT_v, [cpos * B + (b0 + b_loc)])
            ls = plsc.load_gather(lse_v, [iv])
            pk = pick_v[pl.ds(b_loc * LP + k * 16, 16)]
            acc = acc + jnp.where(pos < L, ls - pk, 0.0)
        return acc

    acc = lax.fori_loop(0, BPW, accum, jnp.zeros((16,), jnp.float32))
    tmp_v[...] = acc
    pltpu.sync_copy(tmp_v, part_hbm.at[pl.ds(wid * 32, 16)])
    tmp_v[...] = jnp.zeros((16,), jnp.float32)
    pltpu.sync_copy(tmp_v, part_hbm.at[pl.ds(wid * 32 + 16, 16)])


def kernel(inputs, targets, table):
    inT_flat = inputs.astype(jnp.int32).T.reshape(-1)
    tgt_flat = jnp.pad(targets.astype(jnp.int32), ((0, 0), (0, LP - L))) \
        .reshape(-1)
    tableT = table.T
    table_flat = table.reshape(-1)
    tpad = jnp.pad(table, ((0, VP - V), (0, VP - V)), constant_values=-1e30)
    lse = pl.pallas_call(
        _lse_body,
        out_shape=jax.ShapeDtypeStruct((VP, 1), jnp.float32),
    )(tpad)
    out3d, parts = _sc_gather(tableT, table_flat, inT_flat, tgt_flat,
                              lse.reshape(VP))
    logits = out3d.transpose(2, 0, 1)
    loss11 = pl.pallas_call(
        _loss_body,
        out_shape=jax.ShapeDtypeStruct((1, 1), jnp.float32),
    )(parts.reshape(8, 128))
    return logits, loss11[0, 0]


# speedup vs baseline: 2.0435x; 2.0435x over previous
import functools

import jax
import jax.numpy as jnp
from jax import lax
from jax.experimental import pallas as pl
from jax.experimental.pallas import tpu as pltpu
from jax.experimental.pallas import tpu_sc as plsc

V = 1000
VP = 1024
NC, NS = 2, 16
NW = NC * NS
B = 1024
L = 50
LP = 64
BPW = B // NW
N = B * L
NVT = V // 8


def _lse_body(t_ref, o_ref):
    x = t_ref[...]
    m = jnp.max(x, axis=1, keepdims=True)
    s = jnp.sum(jnp.exp(x - m), axis=1, keepdims=True)
    o_ref[...] = m + jnp.log(s)


def _loss_body(p_ref, o_ref):
    s = jnp.sum(p_ref[...], axis=1, keepdims=True)
    o_ref[...] = jnp.sum(s, axis=0, keepdims=True) * (1.0 / N)


_mesh = plsc.VectorSubcoreMesh(core_axis_name="c", subcore_axis_name="s")


@functools.partial(
    pl.kernel,
    mesh=_mesh,
    compiler_params=pltpu.CompilerParams(
        use_tc_tiling_on_sc=False, needs_layout_passes=False
    ),
    out_type=[
        jax.ShapeDtypeStruct((L, V, B), jnp.float32),
        jax.ShapeDtypeStruct((NW * 32,), jnp.float32),
    ],
    scratch_types=[
        pltpu.VMEM((N,), jnp.int32),
        pltpu.VMEM((BPW * LP,), jnp.int32),
        pltpu.VMEM((VP,), jnp.float32),
        pltpu.VMEM((8, V), jnp.float32),
        pltpu.VMEM((1, 8, B), jnp.float32),
        pltpu.VMEM((1, 8, B), jnp.float32),
        pltpu.VMEM((BPW * LP,), jnp.int32),
        pltpu.VMEM((BPW * LP,), jnp.float32),
        pltpu.VMEM((16,), jnp.float32),
        pltpu.SemaphoreType.DMA,
        pltpu.SemaphoreType.DMA,
        pltpu.SemaphoreType.DMA,
    ],
)
def _sc_gather(tT_hbm, tflat_hbm, inT_hbm, tgt_hbm, lse_hbm, out_hbm, part_hbm,
               inT_v, tgt_v, lse_v, rowb_v, slab_a, slab_b, fidx_v, pick_v,
               tmp_v, sem_wa, sem_wb, sem_p):
    wid = lax.axis_index("s") * NC + lax.axis_index("c")
    pltpu.sync_copy(inT_hbm, inT_v)
    pltpu.sync_copy(tgt_hbm.at[pl.ds(wid * (BPW * LP), BPW * LP)], tgt_v)
    pltpu.sync_copy(lse_hbm, lse_v)
    lane = lax.iota(jnp.int32, 16)

    def fill_and_write(vt, l, slab, sem_w):
        @plsc.parallel_loop(0, B // 16, unroll=4)
        def fill_j(j):
            iv = inT_v[pl.ds(l * B + j * 16, 16)]
            for v_loc in range(8):
                val = plsc.load_gather(
                    rowb_v, [jnp.full((16,), v_loc, jnp.int32), iv])
                slab[0, v_loc, pl.ds(j * 16, 16)] = val
        pltpu.async_copy(
            slab, out_hbm.at[pl.ds(l, 1), pl.ds(vt * 8, 8)], sem_w)

    def wait_write(sem_w):
        pltpu.make_async_copy(
            slab_a, out_hbm.at[pl.ds(0, 1), pl.ds(0, 8)], sem_w).wait()

    for t in range(4):
        vt = wid + 32 * t

        def vt_block():
            pltpu.sync_copy(tT_hbm.at[pl.ds(vt * 8, 8)], rowb_v)

            def pair(u, _):
                @pl.when(jnp.logical_or(jnp.int32(t) > 0, u > 0))
                def _():
                    wait_write(sem_wa)
                fill_and_write(vt, 2 * u, slab_a, sem_wa)

                @pl.when(jnp.logical_or(jnp.int32(t) > 0, u > 0))
                def _():
                    wait_write(sem_wb)
                fill_and_write(vt, 2 * u + 1, slab_b, sem_wb)
                return 0

            lax.fori_loop(0, L // 2, pair, 0)

        if t < 3:
            vt_block()
        else:
            pl.when(wid < NVT - 96)(vt_block)
    wait_write(sem_wa)
    wait_write(sem_wb)

    b0 = wid * BPW

    def build_fidx(b_loc, _):
        for k in range(4):
            pos = lane + (k * 16)
            cpos = jnp.minimum(pos, L - 1)
            iv = plsc.load_gather(inT_v, [cpos * B + (b0 + b_loc)])
            tv = tgt_v[pl.ds(b_loc * LP + k * 16, 16)]
            fidx_v[pl.ds(b_loc * LP + k * 16, 16)] = iv * V + tv
        return 0

    lax.fori_loop(0, BPW, build_fidx, 0)

    def pick_gather(b_loc, _):
        pltpu.async_copy(
            tflat_hbm.at[fidx_v.at[pl.ds(b_loc * LP, LP)]],
            pick_v.at[pl.ds(b_loc * LP, LP)], sem_p).wait()
        return 0

    lax.fori_loop(0, BPW, pick_gather, 0)

    def accum(b_loc, acc):
        for k in range(4):
            pos = lane + (k * 16)
            cpos = jnp.minimum(pos, L - 1)
            iv = plsc.load_gather(inT_v, [cpos * B + (b0 + b_loc)])
            ls = plsc.load_gather(lse_v, [iv])
            pk = pick_v[pl.ds(b_loc * LP + k * 16, 16)]
            acc = acc + jnp.where(pos < L, ls - pk, 0.0)
        return acc

    acc = lax.fori_loop(0, BPW, accum, jnp.zeros((16,), jnp.float32))
    tmp_v[...] = acc
    pltpu.sync_copy(tmp_v, part_hbm.at[pl.ds(wid * 32, 16)])
    tmp_v[...] = jnp.zeros((16,), jnp.float32)
    pltpu.sync_copy(tmp_v, part_hbm.at[pl.ds(wid * 32 + 16, 16)])


def kernel(inputs, targets, table):
    inT_flat = inputs.astype(jnp.int32).T.reshape(-1)
    tgt_flat = jnp.pad(targets.astype(jnp.int32), ((0, 0), (0, LP - L))) \
        .reshape(-1)
    tableT = table.T
    table_flat = table.reshape(-1)
    tpad = jnp.pad(table, ((0, VP - V), (0, VP - V)), constant_values=-1e30)
    lse = pl.pallas_call(
        _lse_body,
        out_shape=jax.ShapeDtypeStruct((VP, 1), jnp.float32),
    )(tpad)
    out3d, parts = _sc_gather(tableT, table_flat, inT_flat, tgt_flat,
                              lse.reshape(VP))
    logits = out3d.transpose(2, 0, 1)
    loss11 = pl.pallas_call(
        _loss_body,
        out_shape=jax.ShapeDtypeStruct((1, 1), jnp.float32),
    )(parts.reshape(8, 128))
    return logits, loss11[0, 0]


# speedup vs baseline: 4.3971x; 2.1518x over previous
import functools

import jax
import jax.numpy as jnp
from jax import lax
from jax.experimental import pallas as pl
from jax.experimental.pallas import tpu as pltpu
from jax.experimental.pallas import tpu_sc as plsc

V = 1000
VP = 1024
NC, NS = 2, 16
NW = NC * NS
B = 1024
L = 50
LP = 64
BPW = B // NW
N = B * L
NVT = V // 8


def _lse_body(t_ref, o_ref):
    x = t_ref[...]
    m = jnp.max(x, axis=1, keepdims=True)
    s = jnp.sum(jnp.exp(x - m), axis=1, keepdims=True)
    o_ref[...] = m + jnp.log(s)


def _loss_body(p_ref, o_ref):
    s = jnp.sum(p_ref[...], axis=1, keepdims=True)
    o_ref[...] = jnp.sum(s, axis=0, keepdims=True) * (1.0 / N)


_mesh = plsc.VectorSubcoreMesh(core_axis_name="c", subcore_axis_name="s")


@functools.partial(
    pl.kernel,
    mesh=_mesh,
    compiler_params=pltpu.CompilerParams(
        use_tc_tiling_on_sc=False, needs_layout_passes=False
    ),
    out_type=[
        jax.ShapeDtypeStruct((L, NVT, 8, 8, 128), jnp.float32),
        jax.ShapeDtypeStruct((NW * 32,), jnp.float32),
    ],
    scratch_types=[
        pltpu.VMEM((N,), jnp.int32),
        pltpu.VMEM((BPW * LP,), jnp.int32),
        pltpu.VMEM((VP,), jnp.float32),
        pltpu.VMEM((8, V), jnp.float32),
        pltpu.VMEM((1, 1, 8, 8, 128), jnp.float32),
        pltpu.VMEM((1, 1, 8, 8, 128), jnp.float32),
        pltpu.VMEM((BPW * LP,), jnp.int32),
        pltpu.VMEM((BPW * LP,), jnp.float32),
        pltpu.VMEM((16,), jnp.float32),
        pltpu.SemaphoreType.DMA,
        pltpu.SemaphoreType.DMA,
        pltpu.SemaphoreType.DMA,
    ],
)
def _sc_gather(tT_hbm, tflat_hbm, inT_hbm, tgt_hbm, lse_hbm, out_hbm, part_hbm,
               inT_v, tgt_v, lse_v, rowb_v, slab_a, slab_b, fidx_v, pick_v,
               tmp_v, sem_wa, sem_wb, sem_p):
    wid = lax.axis_index("s") * NC + lax.axis_index("c")
    pltpu.sync_copy(inT_hbm, inT_v)
    pltpu.sync_copy(tgt_hbm.at[pl.ds(wid * (BPW * LP), BPW * LP)], tgt_v)
    pltpu.sync_copy(lse_hbm, lse_v)
    lane = lax.iota(jnp.int32, 16)

    def fill_and_write(vt, l, slab, sem_w):
        @plsc.parallel_loop(0, B // 16, unroll=4)
        def fill_j(j):
            iv = inT_v[pl.ds(l * B + j * 16, 16)]
            bt = j // 8
            co = (j % 8) * 16
            for v_loc in range(8):
                val = plsc.load_gather(
                    rowb_v, [jnp.full((16,), v_loc, jnp.int32), iv])
                slab[0, 0, bt, v_loc, pl.ds(co, 16)] = val
        pltpu.async_copy(
            slab, out_hbm.at[pl.ds(l, 1), pl.ds(vt, 1)], sem_w)

    def wait_write(sem_w):
        pltpu.make_async_copy(
            slab_a, out_hbm.at[pl.ds(0, 1), pl.ds(0, 1)], sem_w).wait()

    for t in range(4):
        vt = wid + 32 * t

        def vt_block():
            pltpu.sync_copy(tT_hbm.at[pl.ds(vt * 8, 8)], rowb_v)

            def pair(u, _):
                @pl.when(jnp.logical_or(jnp.int32(t) > 0, u > 0))
                def _():
                    wait_write(sem_wa)
                fill_and_write(vt, 2 * u, slab_a, sem_wa)

                @pl.when(jnp.logical_or(jnp.int32(t) > 0, u > 0))
                def _():
                    wait_write(sem_wb)
                fill_and_write(vt, 2 * u + 1, slab_b, sem_wb)
                return 0

            lax.fori_loop(0, L // 2, pair, 0)

        if t < 3:
            vt_block()
        else:
            pl.when(wid < NVT - 96)(vt_block)
    wait_write(sem_wa)
    wait_write(sem_wb)

    b0 = wid * BPW

    def build_fidx(b_loc, _):
        for k in range(4):
            pos = lane + (k * 16)
            cpos = jnp.minimum(pos, L - 1)
            iv = plsc.load_gather(inT_v, [cpos * B + (b0 + b_loc)])
            tv = tgt_v[pl.ds(b_loc * LP + k * 16, 16)]
            fidx_v[pl.ds(b_loc * LP + k * 16, 16)] = iv * V + tv
        return 0

    lax.fori_loop(0, BPW, build_fidx, 0)

    def pick_gather(b_loc, _):
        pltpu.async_copy(
            tflat_hbm.at[fidx_v.at[pl.ds(b_loc * LP, LP)]],
            pick_v.at[pl.ds(b_loc * LP, LP)], sem_p).wait()
        return 0

    lax.fori_loop(0, BPW, pick_gather, 0)

    def accum(b_loc, acc):
        for k in range(4):
            pos = lane + (k * 16)
            cpos = jnp.minimum(pos, L - 1)
            iv = plsc.load_gather(inT_v, [cpos * B + (b0 + b_loc)])
            ls = plsc.load_gather(lse_v, [iv])
            pk = pick_v[pl.ds(b_loc * LP + k * 16, 16)]
            acc = acc + jnp.where(pos < L, ls - pk, 0.0)
        return acc

    acc = lax.fori_loop(0, BPW, accum, jnp.zeros((16,), jnp.float32))
    tmp_v[...] = acc
    pltpu.sync_copy(tmp_v, part_hbm.at[pl.ds(wid * 32, 16)])
    tmp_v[...] = jnp.zeros((16,), jnp.float32)
    pltpu.sync_copy(tmp_v, part_hbm.at[pl.ds(wid * 32 + 16, 16)])


def kernel(inputs, targets, table):
    inT_flat = inputs.astype(jnp.int32).T.reshape(-1)
    tgt_flat = jnp.pad(targets.astype(jnp.int32), ((0, 0), (0, LP - L))) \
        .reshape(-1)
    tableT = table.T
    table_flat = table.reshape(-1)
    tpad = jnp.pad(table, ((0, VP - V), (0, VP - V)), constant_values=-1e30)
    lse = pl.pallas_call(
        _lse_body,
        out_shape=jax.ShapeDtypeStruct((VP, 1), jnp.float32),
    )(tpad)
    out5d, parts = _sc_gather(tableT, table_flat, inT_flat, tgt_flat,
                              lse.reshape(VP))
    logits = out5d.transpose(2, 4, 0, 1, 3).reshape(B, L, V)
    loss11 = pl.pallas_call(
        _loss_body,
        out_shape=jax.ShapeDtypeStruct((1, 1), jnp.float32),
    )(parts.reshape(8, 128))
    return logits, loss11[0, 0]


# speedup vs baseline: 4.4729x; 1.0172x over previous
import functools

import jax
import jax.numpy as jnp
from jax import lax
from jax.experimental import pallas as pl
from jax.experimental.pallas import tpu as pltpu
from jax.experimental.pallas import tpu_sc as plsc

V = 1000
VP = 1024
NC, NS = 2, 16
NW = NC * NS
B = 1024
L = 50
LP = 64
BPW = B // NW
N = B * L
NVT = V // 8


def _lse_body(t_ref, o_ref):
    x = t_ref[...]
    m = jnp.max(x, axis=1, keepdims=True)
    s = jnp.sum(jnp.exp(x - m), axis=1, keepdims=True)
    o_ref[...] = m + jnp.log(s)


def _loss_body(p_ref, o_ref):
    s = jnp.sum(p_ref[...], axis=1, keepdims=True)
    o_ref[...] = jnp.sum(s, axis=0, keepdims=True) * (1.0 / N)


_mesh = plsc.VectorSubcoreMesh(core_axis_name="c", subcore_axis_name="s")


@functools.partial(
    pl.kernel,
    mesh=_mesh,
    compiler_params=pltpu.CompilerParams(
        use_tc_tiling_on_sc=False, needs_layout_passes=False
    ),
    out_type=[
        jax.ShapeDtypeStruct((L, NVT, 8, 8, 128), jnp.float32),
        jax.ShapeDtypeStruct((NW * 32,), jnp.float32),
    ],
    scratch_types=[
        pltpu.VMEM((N,), jnp.int32),
        pltpu.VMEM((BPW * LP,), jnp.int32),
        pltpu.VMEM((VP,), jnp.float32),
        pltpu.VMEM((8, V), jnp.float32),
        pltpu.VMEM((1, 1, 8, 8, 128), jnp.float32),
        pltpu.VMEM((1, 1, 8, 8, 128), jnp.float32),
        pltpu.VMEM((BPW * LP,), jnp.int32),
        pltpu.VMEM((BPW * LP,), jnp.float32),
        pltpu.VMEM((16,), jnp.float32),
        pltpu.SemaphoreType.DMA,
        pltpu.SemaphoreType.DMA,
        pltpu.SemaphoreType.DMA,
    ],
)
def _sc_gather(tT_hbm, tflat_hbm, inT_hbm, tgt_hbm, lse_hbm, out_hbm, part_hbm,
               inT_v, tgt_v, lse_v, rowb_v, slab_a, slab_b, fidx_v, pick_v,
               tmp_v, sem_wa, sem_wb, sem_p):
    wid = lax.axis_index("s") * NC + lax.axis_index("c")
    pltpu.sync_copy(inT_hbm, inT_v)
    pltpu.sync_copy(tgt_hbm.at[pl.ds(wid * (BPW * LP), BPW * LP)], tgt_v)
    pltpu.sync_copy(lse_hbm, lse_v)
    lane = lax.iota(jnp.int32, 16)

    def fill_and_write(vt, l, slab, sem_w):
        @plsc.parallel_loop(0, B // 16, unroll=8)
        def fill_j(j):
            iv = inT_v[pl.ds(l * B + j * 16, 16)]
            bt = j // 8
            co = (j % 8) * 16
            for v_loc in range(8):
                val = plsc.load_gather(
                    rowb_v, [jnp.full((16,), v_loc, jnp.int32), iv])
                slab[0, 0, bt, v_loc, pl.ds(co, 16)] = val
        pltpu.async_copy(
            slab, out_hbm.at[pl.ds(l, 1), pl.ds(vt, 1)], sem_w)

    def wait_write(sem_w):
        pltpu.make_async_copy(
            slab_a, out_hbm.at[pl.ds(0, 1), pl.ds(0, 1)], sem_w).wait()

    for t in range(4):
        vt = wid + 32 * t

        def vt_block():
            pltpu.sync_copy(tT_hbm.at[pl.ds(vt * 8, 8)], rowb_v)

            def pair(u, _):
                @pl.when(jnp.logical_or(jnp.int32(t) > 0, u > 0))
                def _():
                    wait_write(sem_wa)
                fill_and_write(vt, 2 * u, slab_a, sem_wa)

                @pl.when(jnp.logical_or(jnp.int32(t) > 0, u > 0))
                def _():
                    wait_write(sem_wb)
                fill_and_write(vt, 2 * u + 1, slab_b, sem_wb)
                return 0

            lax.fori_loop(0, L // 2, pair, 0)

        if t < 3:
            vt_block()
        else:
            pl.when(wid < NVT - 96)(vt_block)
    wait_write(sem_wa)
    wait_write(sem_wb)

    b0 = wid * BPW

    def build_fidx(b_loc, _):
        for k in range(4):
            pos = lane + (k * 16)
            cpos = jnp.minimum(pos, L - 1)
            iv = plsc.load_gather(inT_v, [cpos * B + (b0 + b_loc)])
            tv = tgt_v[pl.ds(b_loc * LP + k * 16, 16)]
            fidx_v[pl.ds(b_loc * LP + k * 16, 16)] = iv * V + tv
        return 0

    lax.fori_loop(0, BPW, build_fidx, 0)

    def pick_gather(b_loc, _):
        pltpu.async_copy(
            tflat_hbm.at[fidx_v.at[pl.ds(b_loc * LP, LP)]],
            pick_v.at[pl.ds(b_loc * LP, LP)], sem_p).wait()
        return 0

    lax.fori_loop(0, BPW, pick_gather, 0)

    def accum(b_loc, acc):
        for k in range(4):
            pos = lane + (k * 16)
            cpos = jnp.minimum(pos, L - 1)
            iv = plsc.load_gather(inT_v, [cpos * B + (b0 + b_loc)])
            ls = plsc.load_gather(lse_v, [iv])
            pk = pick_v[pl.ds(b_loc * LP + k * 16, 16)]
            acc = acc + jnp.where(pos < L, ls - pk, 0.0)
        return acc

    acc = lax.fori_loop(0, BPW, accum, jnp.zeros((16,), jnp.float32))
    tmp_v[...] = acc
    pltpu.sync_copy(tmp_v, part_hbm.at[pl.ds(wid * 32, 16)])
    tmp_v[...] = jnp.zeros((16,), jnp.float32)
    pltpu.sync_copy(tmp_v, part_hbm.at[pl.ds(wid * 32 + 16, 16)])


def kernel(inputs, targets, table):
    inT_flat = inputs.astype(jnp.int32).T.reshape(-1)
    tgt_flat = jnp.pad(targets.astype(jnp.int32), ((0, 0), (0, LP - L))) \
        .reshape(-1)
    tableT = table.T
    table_flat = table.reshape(-1)
    tpad = jnp.pad(table, ((0, VP - V), (0, VP - V)), constant_values=-1e30)
    lse = pl.pallas_call(
        _lse_body,
        out_shape=jax.ShapeDtypeStruct((VP, 1), jnp.float32),
    )(tpad)
    out5d, parts = _sc_gather(tableT, table_flat, inT_flat, tgt_flat,
                              lse.reshape(VP))
    logits = out5d.transpose(2, 4, 0, 1, 3).reshape(B, L, V)
    loss11 = pl.pallas_call(
        _loss_body,
        out_shape=jax.ShapeDtypeStruct((1, 1), jnp.float32),
    )(parts.reshape(8, 128))
    return logits, loss11[0, 0]
